# baseline (speedup 1.0000x reference)
import jax
import jax.numpy as jnp
from jax.experimental import pallas as pl
from jax.experimental.pallas import tpu as pltpu

D = 256
TM = 512

_DN = (((1,), (1,)), ((), ()))


def _fused_rows(x_ref, win_ref, bin_ref, wcl_ref, bcl_ref, wro_ref, bro_ref,
                o_ref, wc_ref, bc_ref):
    @pl.when(pl.program_id(0) == 0)
    def _precompute():
        wc = jax.lax.dot_general(
            wro_ref[...], wcl_ref[...], (((1,), (0,)), ((), ())),
            preferred_element_type=jnp.float32)
        wc_ref[...] = wc.astype(jnp.bfloat16)
        bc = jax.lax.dot_general(bcl_ref[...], wro_ref[...], _DN,
                                 preferred_element_type=jnp.float32)
        bc_ref[...] = bc + bro_ref[...]

    x = x_ref[...].astype(jnp.bfloat16)
    t = jnp.tanh(jax.lax.dot_general(x, win_ref[...], _DN,
                                     preferred_element_type=jnp.float32)
                 + bin_ref[...])
    o = jax.lax.dot_general(t.astype(jnp.bfloat16), wro_ref[...], _DN,
                            preferred_element_type=jnp.float32)
    o += jax.lax.dot_general(x, wc_ref[...], _DN,
                             preferred_element_type=jnp.float32)
    o_ref[...] = o + bc_ref[...]


def kernel(x_patient, x_se, edge_index, W_in, b_in, W_se, b_se, W_cl, b_cl,
           W_ro, b_ro, Wl_0_pse, bl_0_pse, Wr_0_pse, Wl_0_rev, bl_0_rev,
           Wr_0_rev, Wl_1_pse, bl_1_pse, Wr_1_pse, Wl_1_rev, bl_1_rev,
           Wr_1_rev):
    n_out = x_patient.shape[0] - 1
    grid = (pl.cdiv(n_out, TM),)
    wspec = pl.BlockSpec((D, D), lambda i: (0, 0))
    bspec = pl.BlockSpec((1, D), lambda i: (0, 0))
    out = pl.pallas_call(
        _fused_rows,
        grid=grid,
        in_specs=[
            pl.BlockSpec((TM, D), lambda i: (i, 0)),
            wspec, bspec, wspec, bspec, wspec, bspec,
        ],
        out_specs=pl.BlockSpec((TM, D), lambda i: (i, 0)),
        out_shape=jax.ShapeDtypeStruct((n_out, D), jnp.float32),
        scratch_shapes=[
            pltpu.VMEM((D, D), jnp.bfloat16),
            pltpu.VMEM((1, D), jnp.float32),
        ],
        compiler_params=pltpu.CompilerParams(
            dimension_semantics=("arbitrary",)),
    )(x_patient, W_in.astype(jnp.bfloat16), b_in.reshape(1, D),
      W_cl.astype(jnp.bfloat16), b_cl.reshape(1, D).astype(jnp.bfloat16),
      W_ro.astype(jnp.bfloat16), b_ro.reshape(1, D))
    return out
